# Initial kernel scaffold; baseline (speedup 1.0000x reference)
import jax
import jax.numpy as jnp
from jax.experimental import pallas as pl


def kernel(node_feat, edge_attr, edge_index, batch, Wn1, bn1, Wn2, bn2, We1, be1, We2, be2, Wm1, bm1, Wm2, bm2, Wu1, bu1, Wu2, bu2, Wh1, bh1, Wh2, bh2):
    raise NotImplementedError("write your pallas kernel here")


# speedup vs baseline: 3.8070x; 3.8070x over previous
import functools
import jax
import jax.numpy as jnp
from jax import lax
from jax.experimental import pallas as pl
from jax.experimental.pallas import tpu as pltpu, tpu_sc as plsc

NUM_GRAPHS = 16
NW = 32

_SC_PARAMS = pltpu.CompilerParams(
    needs_layout_passes=False, use_tc_tiling_on_sc=False)
_SC_MESH = plsc.VectorSubcoreMesh(core_axis_name="c", subcore_axis_name="s")


def _leaky(x):
  return jnp.where(x >= 0, x, 0.1 * x)


def _node_mlp_body(x_ref, w1_ref, b1_ref, w2_ref, b2_ref, o_ref):
  h = jnp.dot(x_ref[...], w1_ref[...], preferred_element_type=jnp.float32)
  h = _leaky(h + b1_ref[...])
  o_ref[...] = jnp.dot(h, w2_ref[...],
                       preferred_element_type=jnp.float32) + b2_ref[...]


def _node_mlp(x_pad, Wn1, bn1, Wn2p, bn2p, npad, blk):
  grid = npad // blk
  return pl.pallas_call(
      _node_mlp_body,
      grid=(grid,),
      in_specs=[
          pl.BlockSpec((blk, 128), lambda i: (i, 0)),
          pl.BlockSpec((128, 128), lambda i: (0, 0)),
          pl.BlockSpec((1, 128), lambda i: (0, 0)),
          pl.BlockSpec((128, 16), lambda i: (0, 0)),
          pl.BlockSpec((1, 16), lambda i: (0, 0)),
      ],
      out_specs=pl.BlockSpec((blk, 16), lambda i: (i, 0)),
      out_shape=jax.ShapeDtypeStruct((npad, 16), jnp.float32),
  )(x_pad, Wn1, bn1, Wn2p, bn2p)


def _make_first3_local(E, npad):
  ew = E // NW

  def body(src_hbm, cnt_hbm, slots_hbm, src_v, cnt_v, slots_v):
    cid = lax.axis_index("c")
    sid = lax.axis_index("s")
    w = sid * 2 + cid
    base_e = w * ew
    pltpu.sync_copy(src_hbm.at[pl.ds(base_e, ew)], src_v)

    def zero_body(i, _):
      cnt_v[pl.ds(i * 16, 16)] = jnp.zeros((16,), jnp.int32)
      return 0
    lax.fori_loop(0, npad // 16, zero_body, 0)

    iota = lax.iota(jnp.int32, 16)

    def scan_body(v, _):
      s = src_v[pl.ds(v * 16, 16)]
      cnt1, last = plsc.scan_count(s)
      prior = plsc.load_gather(cnt_v, [s])
      r = prior + cnt1 - 1
      eid = base_e + v * 16 + iota
      slot = s * 3 + jnp.minimum(r, 2)
      plsc.store_scatter(slots_v, [slot], eid, mask=r < 3)
      plsc.store_scatter(cnt_v, [s], prior + cnt1, mask=last)
      return 0
    lax.fori_loop(0, ew // 16, scan_body, 0)

    pltpu.sync_copy(cnt_v, cnt_hbm.at[w])
    pltpu.sync_copy(slots_v, slots_hbm.at[w])

  return pl.kernel(
      body,
      out_type=(jax.ShapeDtypeStruct((NW, npad), jnp.int32),
                jax.ShapeDtypeStruct((NW, 3 * npad), jnp.int32)),
      mesh=_SC_MESH,
      compiler_params=_SC_PARAMS,
      scratch_types=[
          pltpu.VMEM((ew,), jnp.int32),
          pltpu.VMEM((npad,), jnp.int32),
          pltpu.VMEM((3 * npad,), jnp.int32),
      ])


def _make_merge_gather(E, npad):
  npw = npad // NW

  def body(cnt_hbm, slots_hbm, dst_hbm, ea_hbm, nf_hbm,
           er_out, nfr_out, cnt_out,
           loc_cnt, loc_slots, cm_v, m0_v, m1_v, m2_v, nbr_v, rows_v, sem):
    cid = lax.axis_index("c")
    sid = lax.axis_index("s")
    wid = sid * 2 + cid
    nbase = wid * npw

    c1 = pltpu.async_copy(cnt_hbm.at[:, pl.ds(nbase, npw)], loc_cnt, sem)
    c2 = pltpu.async_copy(slots_hbm.at[:, pl.ds(3 * nbase, 3 * npw)],
                          loc_slots, sem)
    c1.wait()
    c2.wait()

    iota = lax.iota(jnp.int32, 16)
    zero = jnp.zeros((16,), jnp.int32)

    def merge_body(nv, _):
      nloc3 = (nv * 16 + iota) * 3
      cnt = zero
      mm0 = zero
      mm1 = zero
      mm2 = zero
      for w in range(NW):
        c = loc_cnt[w, pl.ds(nv * 16, 16)]
        wv = jnp.full((16,), w, jnp.int32)
        b0 = plsc.load_gather(loc_slots, [wv, nloc3])
        b1 = plsc.load_gather(loc_slots, [wv, nloc3 + 1])
        b2 = plsc.load_gather(loc_slots, [wv, nloc3 + 2])
        ce = jnp.minimum(c, 3)
        f0 = cnt == 0
        f1 = cnt == 1
        f2 = cnt == 2
        mm0 = jnp.where(f0 & (ce >= 1), b0, mm0)
        mm1 = jnp.where(f0 & (ce >= 2), b1,
                        jnp.where(f1 & (ce >= 1), b0, mm1))
        mm2 = jnp.where(f0 & (ce >= 3), b2,
                        jnp.where(f1 & (ce >= 2), b1,
                                  jnp.where(f2 & (ce >= 1), b0, mm2)))
        cnt = jnp.minimum(cnt + ce, 3)
      fb = nbase + nv * 16 + iota
      sl = pl.ds(nv * 16, 16)
      m0_v[sl] = jnp.where(cnt >= 1, mm0, fb)
      m1_v[sl] = jnp.where(cnt >= 2, mm1, fb)
      m2_v[sl] = jnp.where(cnt >= 3, mm2, fb)
      cm_v[sl] = cnt
      return 0
    lax.fori_loop(0, npw // 16, merge_body, 0)

    for j, m_v in enumerate((m0_v, m1_v, m2_v)):
      pltpu.async_copy(ea_hbm.at[m_v], rows_v, sem).wait()
      pltpu.sync_copy(rows_v, er_out.at[j, pl.ds(nbase, npw)])
      pltpu.async_copy(dst_hbm.at[m_v], nbr_v, sem).wait()
      pltpu.async_copy(nf_hbm.at[nbr_v], rows_v, sem).wait()
      pltpu.sync_copy(rows_v, nfr_out.at[j, pl.ds(nbase, npw)])
    pltpu.sync_copy(cm_v, cnt_out.at[pl.ds(nbase, npw)])

  return pl.kernel(
      body,
      out_type=(jax.ShapeDtypeStruct((3, npad, 16), jnp.float32),
                jax.ShapeDtypeStruct((3, npad, 16), jnp.float32),
                jax.ShapeDtypeStruct((npad,), jnp.int32)),
      mesh=_SC_MESH,
      compiler_params=_SC_PARAMS,
      scratch_types=[
          pltpu.VMEM((NW, npw), jnp.int32),
          pltpu.VMEM((NW, 3 * npw), jnp.int32),
          pltpu.VMEM((npw,), jnp.int32),
          pltpu.VMEM((npw,), jnp.int32),
          pltpu.VMEM((npw,), jnp.int32),
          pltpu.VMEM((npw,), jnp.int32),
          pltpu.VMEM((npw,), jnp.int32),
          pltpu.VMEM((npw, 16), jnp.float32),
          pltpu.SemaphoreType.DMA,
      ])


def _tail_body(nf_ref, er_ref, nfr_ref, cnt_ref, bat_ref,
               We1_ref, be1_ref, We2p_ref, be2p_ref,
               Wm1ap_ref, Wm1bp_ref, bm1_ref, Wm2p_ref, bm2p3_ref,
               Wu1ap_ref, Wu1bp_ref, bu1_ref, Wu2p_ref, bu2p_ref,
               Wh1p_ref, bh1_ref, Wh2_ref, bh2_ref,
               o_ref, acc_ref):
  i = pl.program_id(0)
  n = pl.num_programs(0)

  s_msg = None
  for j in range(3):
    e_rows = er_ref[j]
    n_rows = nfr_ref[j]
    he = _leaky(jnp.dot(e_rows, We1_ref[...],
                        preferred_element_type=jnp.float32) + be1_ref[...])
    ef = jnp.dot(he, We2p_ref[...],
                 preferred_element_type=jnp.float32) + be2p_ref[...]
    pre = (jnp.dot(ef, Wm1ap_ref[...], preferred_element_type=jnp.float32)
           + jnp.dot(n_rows, Wm1bp_ref[...],
                     preferred_element_type=jnp.float32) + bm1_ref[...])
    lj = _leaky(pre)
    s_msg = lj if s_msg is None else s_msg + lj
  aggr = jnp.dot(s_msg, Wm2p_ref[...],
                 preferred_element_type=jnp.float32) + bm2p3_ref[...]

  nf0 = nf_ref[...]
  hu = _leaky(jnp.dot(nf0, Wu1ap_ref[...], preferred_element_type=jnp.float32)
              + jnp.dot(aggr, Wu1bp_ref[...],
                        preferred_element_type=jnp.float32) + bu1_ref[...])
  nc = jnp.dot(hu, Wu2p_ref[...],
               preferred_element_type=jnp.float32) + bu2p_ref[...]
  valid = cnt_ref[...] >= 3
  nf_final = nf0 + jnp.where(valid, nc, 0.0)

  gids = lax.broadcasted_iota(jnp.int32, (1, NUM_GRAPHS), 1)
  oh = (bat_ref[...] == gids).astype(jnp.float32)
  seg = lax.dot_general(oh, nf_final, (((0,), (0,)), ((), ())),
                        preferred_element_type=jnp.float32)

  @pl.when(i == 0)
  def _():
    acc_ref[...] = jnp.zeros_like(acc_ref)
  acc_ref[...] += seg

  @pl.when(i == n - 1)
  def _():
    hh = _leaky(jnp.dot(acc_ref[...], Wh1p_ref[...],
                        preferred_element_type=jnp.float32) + bh1_ref[...])
    o_ref[...] = jnp.dot(hh, Wh2_ref[...],
                         preferred_element_type=jnp.float32) + bh2_ref[...]


def _tail(nf, er, nfr, cnt2d, bat2d, weights, npad, blk):
  grid = npad // blk
  full = lambda shape: pl.BlockSpec(shape, lambda i: tuple(0 for _ in shape))
  in_specs = [
      pl.BlockSpec((blk, 16), lambda i: (i, 0)),
      pl.BlockSpec((3, blk, 16), lambda i: (0, i, 0)),
      pl.BlockSpec((3, blk, 16), lambda i: (0, i, 0)),
      pl.BlockSpec((blk, 1), lambda i: (i, 0)),
      pl.BlockSpec((blk, 1), lambda i: (i, 0)),
      full((16, 128)), full((1, 128)), full((128, 16)), full((1, 16)),
      full((16, 128)), full((16, 128)), full((1, 128)),
      full((128, 16)), full((1, 16)),
      full((16, 128)), full((16, 128)), full((1, 128)),
      full((128, 16)), full((1, 16)),
      full((16, 128)), full((1, 128)), full((128, 2)), full((1, 2)),
  ]
  return pl.pallas_call(
      _tail_body,
      grid=(grid,),
      in_specs=in_specs,
      out_specs=pl.BlockSpec((NUM_GRAPHS, 2), lambda i: (0, 0)),
      out_shape=jax.ShapeDtypeStruct((NUM_GRAPHS, 2), jnp.float32),
      scratch_shapes=[pltpu.VMEM((NUM_GRAPHS, 16), jnp.float32)],
  )(nf, er, nfr, cnt2d, bat2d, *weights)


def kernel(node_feat, edge_attr, edge_index, batch,
           Wn1, bn1, Wn2, bn2, We1, be1, We2, be2,
           Wm1, bm1, Wm2, bm2, Wu1, bu1, Wu2, bu2,
           Wh1, bh1, Wh2, bh2):
  N, DF = node_feat.shape
  E = edge_attr.shape[0]
  npad = ((N + NW * 16 - 1) // (NW * 16)) * (NW * 16)
  blk = 2048

  src = edge_index[0]
  dst = edge_index[1]

  node_feat_pad = jnp.pad(node_feat.astype(jnp.float32),
                          ((0, npad - N), (0, 0)))
  bat2d = jnp.pad(batch.astype(jnp.int32), (0, npad - N),
                  constant_values=NUM_GRAPHS).reshape(npad, 1)

  f32 = jnp.float32
  Wn2p = jnp.zeros((128, 16), f32).at[:, :3].set(Wn2)
  bn2p = jnp.zeros((1, 16), f32).at[0, :3].set(bn2)
  We2p = jnp.zeros((128, 16), f32).at[:, :3].set(We2)
  be2p = jnp.zeros((1, 16), f32).at[0, :3].set(be2)
  Wm1ap = jnp.zeros((16, 128), f32).at[:3].set(Wm1[:3])
  Wm1bp = jnp.zeros((16, 128), f32).at[:3].set(Wm1[3:6])
  Wm2p = jnp.zeros((128, 16), f32).at[:, :2].set(Wm2)
  bm2p3 = jnp.zeros((1, 16), f32).at[0, :2].set(3.0 * bm2)
  Wu1ap = jnp.zeros((16, 128), f32).at[:3].set(Wu1[:3])
  Wu1bp = jnp.zeros((16, 128), f32).at[:2].set(Wu1[3:5])
  Wu2p = jnp.zeros((128, 16), f32).at[:, :3].set(Wu2)
  bu2p = jnp.zeros((1, 16), f32).at[0, :3].set(bu2)
  Wh1p = jnp.zeros((16, 128), f32).at[:3].set(Wh1)

  nf = _node_mlp(node_feat_pad, Wn1, bn1.reshape(1, 128), Wn2p, bn2p,
                 npad, blk)

  cnt_loc, slots_loc = _make_first3_local(E, npad)(src)
  er, nfr, cntm = _make_merge_gather(E, npad)(
      cnt_loc, slots_loc, dst, edge_attr.astype(jnp.float32), nf)

  weights = (We1, be1.reshape(1, 128), We2p, be2p,
             Wm1ap, Wm1bp, bm1.reshape(1, 128), Wm2p, bm2p3,
             Wu1ap, Wu1bp, bu1.reshape(1, 128), Wu2p, bu2p,
             Wh1p, bh1.reshape(1, 128), Wh2, bh2.reshape(1, 2))
  return _tail(nf, er, nfr, cntm.reshape(npad, 1), bat2d, weights, npad, blk)


# speedup vs baseline: 4.1212x; 1.0825x over previous
import functools
import jax
import jax.numpy as jnp
from jax import lax
from jax.experimental import pallas as pl
from jax.experimental.pallas import tpu as pltpu, tpu_sc as plsc

NUM_GRAPHS = 16
NW = 32

_SC_PARAMS = pltpu.CompilerParams(
    needs_layout_passes=False, use_tc_tiling_on_sc=False)
_SC_MESH = plsc.VectorSubcoreMesh(core_axis_name="c", subcore_axis_name="s")


def _leaky(x):
  return jnp.where(x >= 0, x, 0.1 * x)


def _node_mlp_body(x_ref, w1_ref, b1_ref, w2_ref, b2_ref, o_ref):
  h = jnp.dot(x_ref[...], w1_ref[...], preferred_element_type=jnp.float32)
  h = _leaky(h + b1_ref[...])
  o_ref[...] = jnp.dot(h, w2_ref[...],
                       preferred_element_type=jnp.float32) + b2_ref[...]


def _node_mlp(x_pad, Wn1, bn1, Wn2p, bn2p, npad, blk):
  grid = npad // blk
  return pl.pallas_call(
      _node_mlp_body,
      grid=(grid,),
      in_specs=[
          pl.BlockSpec((blk, 128), lambda i: (i, 0)),
          pl.BlockSpec((128, 128), lambda i: (0, 0)),
          pl.BlockSpec((1, 128), lambda i: (0, 0)),
          pl.BlockSpec((128, 16), lambda i: (0, 0)),
          pl.BlockSpec((1, 16), lambda i: (0, 0)),
      ],
      out_specs=pl.BlockSpec((blk, 16), lambda i: (i, 0)),
      out_shape=jax.ShapeDtypeStruct((npad, 16), jnp.float32),
  )(x_pad, Wn1, bn1, Wn2p, bn2p)


def _make_first3_local(E, npad):
  ew = E // NW

  def body(src_hbm, cnt_hbm, slots_hbm, src_v, cnt_v, slots_v):
    cid = lax.axis_index("c")
    sid = lax.axis_index("s")
    w = sid * 2 + cid
    base_e = w * ew
    pltpu.sync_copy(src_hbm.at[pl.ds(base_e, ew)], src_v)

    def zero_body(i, _):
      cnt_v[pl.ds(i * 16, 16)] = jnp.zeros((16,), jnp.int32)
      return 0
    lax.fori_loop(0, npad // 16, zero_body, 0)

    iota = lax.iota(jnp.int32, 16)

    def scan_body(v, _):
      s = src_v[pl.ds(v * 16, 16)]
      cnt1, last = plsc.scan_count(s)
      prior = plsc.load_gather(cnt_v, [s])
      r = prior + cnt1 - 1
      eid = base_e + v * 16 + iota
      slot = s * 3 + jnp.minimum(r, 2)
      plsc.store_scatter(slots_v, [slot], eid, mask=r < 3)
      plsc.store_scatter(cnt_v, [s], prior + cnt1, mask=last)
      return 0
    lax.fori_loop(0, ew // 16, scan_body, 0)

    pltpu.sync_copy(cnt_v, cnt_hbm.at[w])
    pltpu.sync_copy(slots_v, slots_hbm.at[w])

  return pl.kernel(
      body,
      out_type=(jax.ShapeDtypeStruct((NW, npad), jnp.int32),
                jax.ShapeDtypeStruct((NW, 3 * npad), jnp.int32)),
      mesh=_SC_MESH,
      compiler_params=_SC_PARAMS,
      scratch_types=[
          pltpu.VMEM((ew,), jnp.int32),
          pltpu.VMEM((npad,), jnp.int32),
          pltpu.VMEM((3 * npad,), jnp.int32),
      ])


def _make_merge_gather(E, npad):
  npw = npad // NW

  def body(cnt_hbm, slots_hbm, dst_hbm, ea_hbm, nf_hbm,
           packed_out, cnt_out,
           loc_cnt, loc_slots, cm_v, m0_v, m1_v, m2_v, nbr_v, rows_v, z_v,
           sem):
    cid = lax.axis_index("c")
    sid = lax.axis_index("s")
    wid = sid * 2 + cid
    nbase = wid * npw

    c1 = pltpu.async_copy(cnt_hbm.at[:, pl.ds(nbase, npw)], loc_cnt, sem)
    c2 = pltpu.async_copy(slots_hbm.at[:, pl.ds(3 * nbase, 3 * npw)],
                          loc_slots, sem)
    c1.wait()
    c2.wait()

    iota = lax.iota(jnp.int32, 16)
    zero = jnp.zeros((16,), jnp.int32)

    def merge_body(nv, _):
      nloc3 = (nv * 16 + iota) * 3
      cnt = zero
      mm0 = zero
      mm1 = zero
      mm2 = zero
      for w in range(NW):
        c = loc_cnt[w, pl.ds(nv * 16, 16)]
        wv = jnp.full((16,), w, jnp.int32)
        b0 = plsc.load_gather(loc_slots, [wv, nloc3])
        b1 = plsc.load_gather(loc_slots, [wv, nloc3 + 1])
        b2 = plsc.load_gather(loc_slots, [wv, nloc3 + 2])
        ce = jnp.minimum(c, 3)
        f0 = cnt == 0
        f1 = cnt == 1
        f2 = cnt == 2
        mm0 = jnp.where(f0 & (ce >= 1), b0, mm0)
        mm1 = jnp.where(f0 & (ce >= 2), b1,
                        jnp.where(f1 & (ce >= 1), b0, mm1))
        mm2 = jnp.where(f0 & (ce >= 3), b2,
                        jnp.where(f1 & (ce >= 2), b1,
                                  jnp.where(f2 & (ce >= 1), b0, mm2)))
        cnt = jnp.minimum(cnt + ce, 3)
      fb = nbase + nv * 16 + iota
      sl = pl.ds(nv * 16, 16)
      m0_v[sl] = jnp.where(cnt >= 1, mm0, fb)
      m1_v[sl] = jnp.where(cnt >= 2, mm1, fb)
      m2_v[sl] = jnp.where(cnt >= 3, mm2, fb)
      cm_v[sl] = cnt
      return 0
    lax.fori_loop(0, npw // 16, merge_body, 0)

    def zf_body(i, _):
      z_v[i, pl.ds(0, 16)] = jnp.zeros((16,), jnp.float32)
      z_v[i, pl.ds(16, 16)] = jnp.zeros((16,), jnp.float32)
      return 0
    lax.fori_loop(0, npw, zf_body, 0)
    pltpu.sync_copy(z_v, packed_out.at[pl.ds(nbase, npw), pl.ds(96, 32)])
    for j, m_v in enumerate((m0_v, m1_v, m2_v)):
      pltpu.async_copy(ea_hbm.at[m_v], rows_v, sem).wait()
      pltpu.sync_copy(rows_v,
                      packed_out.at[pl.ds(nbase, npw), pl.ds(j * 16, 16)])
      pltpu.async_copy(dst_hbm.at[m_v], nbr_v, sem).wait()
      pltpu.async_copy(nf_hbm.at[nbr_v], rows_v, sem).wait()
      pltpu.sync_copy(rows_v,
                      packed_out.at[pl.ds(nbase, npw), pl.ds(48 + j * 16, 16)])
    pltpu.sync_copy(cm_v, cnt_out.at[pl.ds(nbase, npw)])

  return pl.kernel(
      body,
      out_type=(jax.ShapeDtypeStruct((npad, 128), jnp.float32),
                jax.ShapeDtypeStruct((npad,), jnp.int32)),
      mesh=_SC_MESH,
      compiler_params=_SC_PARAMS,
      scratch_types=[
          pltpu.VMEM((NW, npw), jnp.int32),
          pltpu.VMEM((NW, 3 * npw), jnp.int32),
          pltpu.VMEM((npw,), jnp.int32),
          pltpu.VMEM((npw,), jnp.int32),
          pltpu.VMEM((npw,), jnp.int32),
          pltpu.VMEM((npw,), jnp.int32),
          pltpu.VMEM((npw,), jnp.int32),
          pltpu.VMEM((npw, 16), jnp.float32),
          pltpu.VMEM((npw, 32), jnp.float32),
          pltpu.SemaphoreType.DMA,
      ])


def _tail_body(nf_ref, pk_ref, cnt_ref, bat_ref,
               We1j_ref, be1_ref, We2p_ref, be2p_ref,
               Wm1ap_ref, Wm1bj_ref, bm1_ref, Wm2p_ref, bm2p3_ref,
               Wu1ap_ref, Wu1bp_ref, bu1_ref, Wu2p_ref, bu2p_ref,
               Wh1p_ref, bh1_ref, Wh2_ref, bh2_ref,
               o_ref, acc_ref):
  i = pl.program_id(0)
  n = pl.num_programs(0)

  pk = pk_ref[...]
  s_msg = None
  for j in range(3):
    he = _leaky(jnp.dot(pk, We1j_ref[j],
                        preferred_element_type=jnp.float32) + be1_ref[...])
    ef = jnp.dot(he, We2p_ref[...],
                 preferred_element_type=jnp.float32) + be2p_ref[...]
    pre = (jnp.dot(ef, Wm1ap_ref[...], preferred_element_type=jnp.float32)
           + jnp.dot(pk, Wm1bj_ref[j],
                     preferred_element_type=jnp.float32) + bm1_ref[...])
    lj = _leaky(pre)
    s_msg = lj if s_msg is None else s_msg + lj
  aggr = jnp.dot(s_msg, Wm2p_ref[...],
                 preferred_element_type=jnp.float32) + bm2p3_ref[...]

  nf0 = nf_ref[...]
  hu = _leaky(jnp.dot(nf0, Wu1ap_ref[...], preferred_element_type=jnp.float32)
              + jnp.dot(aggr, Wu1bp_ref[...],
                        preferred_element_type=jnp.float32) + bu1_ref[...])
  nc = jnp.dot(hu, Wu2p_ref[...],
               preferred_element_type=jnp.float32) + bu2p_ref[...]
  valid = cnt_ref[...] >= 3
  nf_final = nf0 + jnp.where(valid, nc, 0.0)

  gids = lax.broadcasted_iota(jnp.int32, (1, NUM_GRAPHS), 1)
  oh = (bat_ref[...] == gids).astype(jnp.float32)
  seg = lax.dot_general(oh, nf_final, (((0,), (0,)), ((), ())),
                        preferred_element_type=jnp.float32)

  @pl.when(i == 0)
  def _():
    acc_ref[...] = jnp.zeros_like(acc_ref)
  acc_ref[...] += seg

  @pl.when(i == n - 1)
  def _():
    hh = _leaky(jnp.dot(acc_ref[...], Wh1p_ref[...],
                        preferred_element_type=jnp.float32) + bh1_ref[...])
    o_ref[...] = jnp.dot(hh, Wh2_ref[...],
                         preferred_element_type=jnp.float32) + bh2_ref[...]


def _tail(nf, packed, cnt2d, bat2d, weights, npad, blk):
  grid = npad // blk
  full = lambda shape: pl.BlockSpec(shape, lambda i: tuple(0 for _ in shape))
  in_specs = [
      pl.BlockSpec((blk, 16), lambda i: (i, 0)),
      pl.BlockSpec((blk, 128), lambda i: (i, 0)),
      pl.BlockSpec((blk, 1), lambda i: (i, 0)),
      pl.BlockSpec((blk, 1), lambda i: (i, 0)),
      full((3, 128, 128)), full((1, 128)), full((128, 16)), full((1, 16)),
      full((16, 128)), full((3, 128, 128)), full((1, 128)),
      full((128, 16)), full((1, 16)),
      full((16, 128)), full((16, 128)), full((1, 128)),
      full((128, 16)), full((1, 16)),
      full((16, 128)), full((1, 128)), full((128, 2)), full((1, 2)),
  ]
  return pl.pallas_call(
      _tail_body,
      grid=(grid,),
      in_specs=in_specs,
      out_specs=pl.BlockSpec((NUM_GRAPHS, 2), lambda i: (0, 0)),
      out_shape=jax.ShapeDtypeStruct((NUM_GRAPHS, 2), jnp.float32),
      scratch_shapes=[pltpu.VMEM((NUM_GRAPHS, 16), jnp.float32)],
  )(nf, packed, cnt2d, bat2d, *weights)


def kernel(node_feat, edge_attr, edge_index, batch,
           Wn1, bn1, Wn2, bn2, We1, be1, We2, be2,
           Wm1, bm1, Wm2, bm2, Wu1, bu1, Wu2, bu2,
           Wh1, bh1, Wh2, bh2):
  N, DF = node_feat.shape
  E = edge_attr.shape[0]
  npad = ((N + NW * 16 - 1) // (NW * 16)) * (NW * 16)
  blk = 2048

  src = edge_index[0]
  dst = edge_index[1]

  node_feat_pad = jnp.pad(node_feat.astype(jnp.float32),
                          ((0, npad - N), (0, 0)))
  bat2d = jnp.pad(batch.astype(jnp.int32), (0, npad - N),
                  constant_values=NUM_GRAPHS).reshape(npad, 1)

  f32 = jnp.float32
  Wn2p = jnp.zeros((128, 16), f32).at[:, :3].set(Wn2)
  bn2p = jnp.zeros((1, 16), f32).at[0, :3].set(bn2)
  We2p = jnp.zeros((128, 16), f32).at[:, :3].set(We2)
  be2p = jnp.zeros((1, 16), f32).at[0, :3].set(be2)
  Wm1ap = jnp.zeros((16, 128), f32).at[:3].set(Wm1[:3])
  We1j = jnp.stack([jnp.zeros((128, 128), f32).at[j * 16:j * 16 + 16].set(We1)
                    for j in range(3)])
  Wm1bj = jnp.stack(
      [jnp.zeros((128, 128), f32).at[48 + j * 16:48 + j * 16 + 3].set(Wm1[3:6])
       for j in range(3)])
  Wm2p = jnp.zeros((128, 16), f32).at[:, :2].set(Wm2)
  bm2p3 = jnp.zeros((1, 16), f32).at[0, :2].set(3.0 * bm2)
  Wu1ap = jnp.zeros((16, 128), f32).at[:3].set(Wu1[:3])
  Wu1bp = jnp.zeros((16, 128), f32).at[:2].set(Wu1[3:5])
  Wu2p = jnp.zeros((128, 16), f32).at[:, :3].set(Wu2)
  bu2p = jnp.zeros((1, 16), f32).at[0, :3].set(bu2)
  Wh1p = jnp.zeros((16, 128), f32).at[:3].set(Wh1)

  nf = _node_mlp(node_feat_pad, Wn1, bn1.reshape(1, 128), Wn2p, bn2p,
                 npad, blk)

  cnt_loc, slots_loc = _make_first3_local(E, npad)(src)
  packed, cntm = _make_merge_gather(E, npad)(
      cnt_loc, slots_loc, dst, edge_attr.astype(jnp.float32), nf)

  weights = (We1j, be1.reshape(1, 128), We2p, be2p,
             Wm1ap, Wm1bj, bm1.reshape(1, 128), Wm2p, bm2p3,
             Wu1ap, Wu1bp, bu1.reshape(1, 128), Wu2p, bu2p,
             Wh1p, bh1.reshape(1, 128), Wh2, bh2.reshape(1, 2))
  return _tail(nf, packed, cntm.reshape(npad, 1), bat2d, weights, npad, blk)


# speedup vs baseline: 6.2698x; 1.5214x over previous
import functools
import jax
import jax.numpy as jnp
from jax import lax
from jax.experimental import pallas as pl
from jax.experimental.pallas import tpu as pltpu, tpu_sc as plsc

NUM_GRAPHS = 16
NW = 32

_SC_PARAMS = pltpu.CompilerParams(
    needs_layout_passes=False, use_tc_tiling_on_sc=False)
_SC_MESH = plsc.VectorSubcoreMesh(core_axis_name="c", subcore_axis_name="s")


def _leaky(x):
  return jnp.where(x >= 0, x, 0.1 * x)


def _node_mlp_body(x_ref, w1_ref, b1_ref, w2_ref, b2_ref, o_ref):
  h = jnp.dot(x_ref[...], w1_ref[...], preferred_element_type=jnp.float32)
  h = _leaky(h + b1_ref[...])
  o_ref[...] = jnp.dot(h, w2_ref[...],
                       preferred_element_type=jnp.float32) + b2_ref[...]


def _node_mlp(x_pad, Wn1, bn1, Wn2p, bn2p, npad, blk):
  grid = npad // blk
  return pl.pallas_call(
      _node_mlp_body,
      grid=(grid,),
      in_specs=[
          pl.BlockSpec((blk, 128), lambda i: (i, 0)),
          pl.BlockSpec((128, 128), lambda i: (0, 0)),
          pl.BlockSpec((1, 128), lambda i: (0, 0)),
          pl.BlockSpec((128, 16), lambda i: (0, 0)),
          pl.BlockSpec((1, 16), lambda i: (0, 0)),
      ],
      out_specs=pl.BlockSpec((blk, 16), lambda i: (i, 0)),
      out_shape=jax.ShapeDtypeStruct((npad, 16), jnp.float32),
  )(x_pad, Wn1, bn1, Wn2p, bn2p)


def _make_first3_local(E, npad):
  ew = E // NW

  def body(src_hbm, cnt_hbm, slots_hbm, src_v, cnt_v, slots_v):
    cid = lax.axis_index("c")
    sid = lax.axis_index("s")
    w = sid * 2 + cid
    base_e = w * ew
    pltpu.sync_copy(src_hbm.at[pl.ds(base_e, ew)], src_v)

    def zero_body(i, _):
      cnt_v[pl.ds(i * 16, 16)] = jnp.zeros((16,), jnp.int32)
      return 0
    lax.fori_loop(0, npad // 16, zero_body, 0)

    iota = lax.iota(jnp.int32, 16)

    def scan_body(v, _):
      s = src_v[pl.ds(v * 16, 16)]
      cnt1, last = plsc.scan_count(s)
      prior = plsc.load_gather(cnt_v, [s])
      r = prior + cnt1 - 1
      eid = base_e + v * 16 + iota
      slot = s * 3 + jnp.minimum(r, 2)
      plsc.store_scatter(slots_v, [slot], eid, mask=r < 3)
      plsc.store_scatter(cnt_v, [s], prior + cnt1, mask=last)
      return 0
    lax.fori_loop(0, ew // 16, scan_body, 0)

    pltpu.sync_copy(cnt_v, cnt_hbm.at[w])
    pltpu.sync_copy(slots_v, slots_hbm.at[w])

  return pl.kernel(
      body,
      out_type=(jax.ShapeDtypeStruct((NW, npad), jnp.int32),
                jax.ShapeDtypeStruct((NW, 3 * npad), jnp.int32)),
      mesh=_SC_MESH,
      compiler_params=_SC_PARAMS,
      scratch_types=[
          pltpu.VMEM((ew,), jnp.int32),
          pltpu.VMEM((npad,), jnp.int32),
          pltpu.VMEM((3 * npad,), jnp.int32),
      ])


def _make_merge_gather(E, npad):
  npw = npad // NW
  tcols = E // 128

  def body(cnt_hbm, slots_hbm, dst_hbm, ea_hbm, nf_hbm,
           packed_out, cnt_out,
           loc_cnt, loc_slots, cm_v, m0_v, m1_v, m2_v, nbr_v, rows_v,
           pk_v, idx_v, ev_v, sem):
    cid = lax.axis_index("c")
    sid = lax.axis_index("s")
    wid = sid * 2 + cid
    nbase = wid * npw

    c1 = pltpu.async_copy(cnt_hbm.at[:, pl.ds(nbase, npw)], loc_cnt, sem)
    c2 = pltpu.async_copy(slots_hbm.at[:, pl.ds(3 * nbase, 3 * npw)],
                          loc_slots, sem)
    c1.wait()
    c2.wait()

    iota = lax.iota(jnp.int32, 16)
    zero = jnp.zeros((16,), jnp.int32)

    def merge_body(nv, _):
      nloc3 = (nv * 16 + iota) * 3
      cnt = zero
      mm0 = zero
      mm1 = zero
      mm2 = zero
      for w in range(NW):
        c = loc_cnt[w, pl.ds(nv * 16, 16)]
        wv = jnp.full((16,), w, jnp.int32)
        b0 = plsc.load_gather(loc_slots, [wv, nloc3])
        b1 = plsc.load_gather(loc_slots, [wv, nloc3 + 1])
        b2 = plsc.load_gather(loc_slots, [wv, nloc3 + 2])
        ce = jnp.minimum(c, 3)
        f0 = cnt == 0
        f1 = cnt == 1
        f2 = cnt == 2
        mm0 = jnp.where(f0 & (ce >= 1), b0, mm0)
        mm1 = jnp.where(f0 & (ce >= 2), b1,
                        jnp.where(f1 & (ce >= 1), b0, mm1))
        mm2 = jnp.where(f0 & (ce >= 3), b2,
                        jnp.where(f1 & (ce >= 2), b1,
                                  jnp.where(f2 & (ce >= 1), b0, mm2)))
        cnt = jnp.minimum(cnt + ce, 3)
      fb = nbase + nv * 16 + iota
      sl = pl.ds(nv * 16, 16)
      m0_v[sl] = jnp.where(cnt >= 1, mm0, fb)
      m1_v[sl] = jnp.where(cnt >= 2, mm1, fb)
      m2_v[sl] = jnp.where(cnt >= 3, mm2, fb)
      cm_v[sl] = cnt
      return 0
    lax.fori_loop(0, npw // 16, merge_body, 0)

    zeros16f = jnp.zeros((16,), jnp.float32)

    def z_body(t, _):
      pk_v[t, pl.ds(96, 16)] = zeros16f
      pk_v[t, pl.ds(112, 16)] = zeros16f
      return 0
    lax.fori_loop(0, npw, z_body, 0)

    for j, m_v in enumerate((m0_v, m1_v, m2_v)):
      def gidx_body(t, _):
        m = m_v[pl.ds(t * 16, 16)]
        g = lax.shift_right_logical(m, 7) * 1024 + (m & 127)
        for c in range(16):
          fc = (c // 8) * (tcols * 1024) + (c % 8) * 128
          idx_v[pl.ds(c * npw + t * 16, 16)] = g + fc
        return 0
      lax.fori_loop(0, npw // 16, gidx_body, 0)
      pltpu.async_copy(ea_hbm.at[idx_v], ev_v, sem).wait()

      def esc_body(t, _):
        rows = t * 16 + iota
        for c in range(16):
          v = ev_v[pl.ds(c * npw + t * 16, 16)]
          plsc.store_scatter(pk_v, [rows, jnp.full((16,), j * 16 + c,
                                                   jnp.int32)], v)
        return 0
      lax.fori_loop(0, npw // 16, esc_body, 0)

      pltpu.async_copy(dst_hbm.at[m_v], nbr_v, sem).wait()
      pltpu.async_copy(nf_hbm.at[nbr_v], rows_v, sem).wait()

      def nfc_body(t, _):
        pk_v[t, pl.ds(48 + j * 16, 16)] = rows_v[t, pl.ds(0, 16)]
        return 0
      lax.fori_loop(0, npw, nfc_body, 0)

    pltpu.sync_copy(pk_v, packed_out.at[pl.ds(nbase, npw)])
    pltpu.sync_copy(cm_v, cnt_out.at[pl.ds(nbase, npw)])

  return pl.kernel(
      body,
      out_type=(jax.ShapeDtypeStruct((npad, 128), jnp.float32),
                jax.ShapeDtypeStruct((npad,), jnp.int32)),
      mesh=_SC_MESH,
      compiler_params=_SC_PARAMS,
      scratch_types=[
          pltpu.VMEM((NW, npw), jnp.int32),
          pltpu.VMEM((NW, 3 * npw), jnp.int32),
          pltpu.VMEM((npw,), jnp.int32),
          pltpu.VMEM((npw,), jnp.int32),
          pltpu.VMEM((npw,), jnp.int32),
          pltpu.VMEM((npw,), jnp.int32),
          pltpu.VMEM((npw,), jnp.int32),
          pltpu.VMEM((npw, 16), jnp.float32),
          pltpu.VMEM((npw, 128), jnp.float32),
          pltpu.VMEM((16 * npw,), jnp.int32),
          pltpu.VMEM((16 * npw,), jnp.float32),
          pltpu.SemaphoreType.DMA,
      ])


def _tail_body(nf_ref, pk_ref, cnt_ref, bat_ref,
               We1j_ref, be1_ref, We2p_ref, be2p_ref,
               Wm1ap_ref, Wm1bj_ref, bm1_ref, Wm2p_ref, bm2p3_ref,
               Wu1ap_ref, Wu1bp_ref, bu1_ref, Wu2p_ref, bu2p_ref,
               Wh1p_ref, bh1_ref, Wh2_ref, bh2_ref,
               o_ref, acc_ref):
  i = pl.program_id(0)
  n = pl.num_programs(0)

  pk = pk_ref[...]
  s_msg = None
  for j in range(3):
    he = _leaky(jnp.dot(pk, We1j_ref[j],
                        preferred_element_type=jnp.float32) + be1_ref[...])
    ef = jnp.dot(he, We2p_ref[...],
                 preferred_element_type=jnp.float32) + be2p_ref[...]
    pre = (jnp.dot(ef, Wm1ap_ref[...], preferred_element_type=jnp.float32)
           + jnp.dot(pk, Wm1bj_ref[j],
                     preferred_element_type=jnp.float32) + bm1_ref[...])
    lj = _leaky(pre)
    s_msg = lj if s_msg is None else s_msg + lj
  aggr = jnp.dot(s_msg, Wm2p_ref[...],
                 preferred_element_type=jnp.float32) + bm2p3_ref[...]

  nf0 = nf_ref[...]
  hu = _leaky(jnp.dot(nf0, Wu1ap_ref[...], preferred_element_type=jnp.float32)
              + jnp.dot(aggr, Wu1bp_ref[...],
                        preferred_element_type=jnp.float32) + bu1_ref[...])
  nc = jnp.dot(hu, Wu2p_ref[...],
               preferred_element_type=jnp.float32) + bu2p_ref[...]
  valid = cnt_ref[...] >= 3
  nf_final = nf0 + jnp.where(valid, nc, 0.0)

  gids = lax.broadcasted_iota(jnp.int32, (1, NUM_GRAPHS), 1)
  oh = (bat_ref[...] == gids).astype(jnp.float32)
  seg = lax.dot_general(oh, nf_final, (((0,), (0,)), ((), ())),
                        preferred_element_type=jnp.float32)

  @pl.when(i == 0)
  def _():
    acc_ref[...] = jnp.zeros_like(acc_ref)
  acc_ref[...] += seg

  @pl.when(i == n - 1)
  def _():
    hh = _leaky(jnp.dot(acc_ref[...], Wh1p_ref[...],
                        preferred_element_type=jnp.float32) + bh1_ref[...])
    o_ref[...] = jnp.dot(hh, Wh2_ref[...],
                         preferred_element_type=jnp.float32) + bh2_ref[...]


def _tail(nf, packed, cnt2d, bat2d, weights, npad, blk):
  grid = npad // blk
  full = lambda shape: pl.BlockSpec(shape, lambda i: tuple(0 for _ in shape))
  in_specs = [
      pl.BlockSpec((blk, 16), lambda i: (i, 0)),
      pl.BlockSpec((blk, 128), lambda i: (i, 0)),
      pl.BlockSpec((blk, 1), lambda i: (i, 0)),
      pl.BlockSpec((blk, 1), lambda i: (i, 0)),
      full((3, 128, 128)), full((1, 128)), full((128, 16)), full((1, 16)),
      full((16, 128)), full((3, 128, 128)), full((1, 128)),
      full((128, 16)), full((1, 16)),
      full((16, 128)), full((16, 128)), full((1, 128)),
      full((128, 16)), full((1, 16)),
      full((16, 128)), full((1, 128)), full((128, 2)), full((1, 2)),
  ]
  return pl.pallas_call(
      _tail_body,
      grid=(grid,),
      in_specs=in_specs,
      out_specs=pl.BlockSpec((NUM_GRAPHS, 2), lambda i: (0, 0)),
      out_shape=jax.ShapeDtypeStruct((NUM_GRAPHS, 2), jnp.float32),
      scratch_shapes=[pltpu.VMEM((NUM_GRAPHS, 16), jnp.float32)],
  )(nf, packed, cnt2d, bat2d, *weights)


def kernel(node_feat, edge_attr, edge_index, batch,
           Wn1, bn1, Wn2, bn2, We1, be1, We2, be2,
           Wm1, bm1, Wm2, bm2, Wu1, bu1, Wu2, bu2,
           Wh1, bh1, Wh2, bh2):
  N, DF = node_feat.shape
  E = edge_attr.shape[0]
  npad = ((N + NW * 16 - 1) // (NW * 16)) * (NW * 16)
  blk = 2048

  src = edge_index[0]
  dst = edge_index[1]

  node_feat_pad = jnp.pad(node_feat.astype(jnp.float32),
                          ((0, npad - N), (0, 0)))
  bat2d = jnp.pad(batch.astype(jnp.int32), (0, npad - N),
                  constant_values=NUM_GRAPHS).reshape(npad, 1)

  f32 = jnp.float32
  Wn2p = jnp.zeros((128, 16), f32).at[:, :3].set(Wn2)
  bn2p = jnp.zeros((1, 16), f32).at[0, :3].set(bn2)
  We2p = jnp.zeros((128, 16), f32).at[:, :3].set(We2)
  be2p = jnp.zeros((1, 16), f32).at[0, :3].set(be2)
  Wm1ap = jnp.zeros((16, 128), f32).at[:3].set(Wm1[:3])
  We1j = jnp.stack([jnp.zeros((128, 128), f32).at[j * 16:j * 16 + 16].set(We1)
                    for j in range(3)])
  Wm1bj = jnp.stack(
      [jnp.zeros((128, 128), f32).at[48 + j * 16:48 + j * 16 + 3].set(Wm1[3:6])
       for j in range(3)])
  Wm2p = jnp.zeros((128, 16), f32).at[:, :2].set(Wm2)
  bm2p3 = jnp.zeros((1, 16), f32).at[0, :2].set(3.0 * bm2)
  Wu1ap = jnp.zeros((16, 128), f32).at[:3].set(Wu1[:3])
  Wu1bp = jnp.zeros((16, 128), f32).at[:2].set(Wu1[3:5])
  Wu2p = jnp.zeros((128, 16), f32).at[:, :3].set(Wu2)
  bu2p = jnp.zeros((1, 16), f32).at[0, :3].set(bu2)
  Wh1p = jnp.zeros((16, 128), f32).at[:3].set(Wh1)

  nf = _node_mlp(node_feat_pad, Wn1, bn1.reshape(1, 128), Wn2p, bn2p,
                 npad, blk)

  tcols = E // 128
  ea_lin = (edge_attr.astype(jnp.float32).T
            .reshape(2, 8, tcols, 128)
            .transpose(0, 2, 1, 3)
            .reshape(E * 16))

  cnt_loc, slots_loc = _make_first3_local(E, npad)(src)
  packed, cntm = _make_merge_gather(E, npad)(
      cnt_loc, slots_loc, dst, ea_lin, nf)

  weights = (We1j, be1.reshape(1, 128), We2p, be2p,
             Wm1ap, Wm1bj, bm1.reshape(1, 128), Wm2p, bm2p3,
             Wu1ap, Wu1bp, bu1.reshape(1, 128), Wu2p, bu2p,
             Wh1p, bh1.reshape(1, 128), Wh2, bh2.reshape(1, 2))
  return _tail(nf, packed, cntm.reshape(npad, 1), bat2d, weights, npad, blk)
